# baseline (speedup 1.0000x reference)
import jax
import jax.numpy as jnp
import numpy as np
from jax.experimental import pallas as pl
from jax.experimental.pallas import tpu as pltpu

B = 16
C = 1000
T = 12
A = 6
H = 128
Z = 32
N = B * C
TOPK = 100
ITERS = 2
MIN_A = -1.0
MAX_A = 1.0
G = 4
CB = B // G
NC = CB * C

_SIGN = np.array(1 << 31, dtype=np.uint32).view(np.int32).item()


def _fkey(x):
    return x ^ (jax.lax.shift_right_arithmetic(x, 31) & jnp.int32(0x7FFFFFFF))


def _cem_kernel(beliefT_ref, WbT_ref, WaT_ref, WsT_ref, wrT_ref,
                eps_ref, out_ref, objrow_s, acts_s, mean_s, std_s):
    it = pl.program_id(0)
    ci = pl.program_id(1)

    @pl.when(jnp.logical_and(it == 0, ci == 0))
    def _init():
        mean_s[...] = jnp.zeros_like(mean_s)
        std_s[...] = jnp.ones_like(std_s)

    WbT = WbT_ref[...]
    WaT = WaT_ref[...]
    WsT = WsT_ref[...]
    wrT = wrT_ref[...]

    b = jnp.concatenate(
        [jnp.broadcast_to(beliefT_ref[ci * CB + j], (H, C))
         for j in range(CB)], axis=1)
    objacc = jnp.zeros((1, NC), dtype=jnp.float32)
    for t in range(T):
        eps_t = eps_ref[0, t, :, 0, 0]
        pieces = [jnp.clip(
            jnp.broadcast_to(mean_s[ci * CB + j, t], (A, C))
            + jnp.broadcast_to(std_s[ci * CB + j, t], (A, C))
            * eps_t[:, j * C:(j + 1) * C],
            MIN_A, MAX_A) for j in range(CB)]
        if t == 0:
            for j in range(CB):
                acts_s[t, ci * CB + j] = pieces[j]
        else:
            @pl.when(it < ITERS - 1)
            def _store_acts():
                for j in range(CB):
                    acts_s[t, ci * CB + j] = pieces[j]
        act = jnp.concatenate(pieces, axis=1)
        pre = (jnp.dot(WbT, b, preferred_element_type=jnp.float32)
               + jnp.dot(WaT, act, preferred_element_type=jnp.float32))
        b = jnp.tanh(pre)
        s = jnp.tanh(jnp.dot(WsT, b, preferred_element_type=jnp.float32))
        feats = jnp.concatenate([b, s], axis=0)
        objacc = objacc + jnp.dot(wrT, feats,
                                  preferred_element_type=jnp.float32)
    objrow_s[ci] = objacc

    @pl.when(ci == G - 1)
    def _select_and_refit():
        obj2 = jnp.concatenate(
            [objrow_s[bb // CB, :, (bb % CB) * C:(bb % CB + 1) * C]
             for bb in range(B)], axis=0)
        obj3 = jnp.concatenate(
            [objrow_s[bb // CB, :, (bb % CB) * C:(bb % CB + 1) * C]
             .reshape(1, 1, C) for bb in range(B)], axis=0)

        key = _fkey(jax.lax.bitcast_convert_type(obj2, jnp.int32))
        theta_u = jnp.zeros((B, 1), dtype=jnp.int32)
        for bit in range(31, -1, -1):
            bit_val = np.array(1 << bit, dtype=np.uint32).view(np.int32).item()
            cand_u = theta_u | jnp.int32(bit_val)
            cand_s = cand_u ^ jnp.int32(_SIGN)
            cnt = jnp.sum((key >= cand_s).astype(jnp.int32), axis=1,
                          keepdims=True)
            theta_u = jnp.where(cnt >= TOPK, cand_u, theta_u)
        theta_f = jax.lax.bitcast_convert_type(
            _fkey(theta_u ^ jnp.int32(_SIGN)), jnp.float32)
        theta3 = jnp.concatenate(
            [theta_f[bb:bb + 1, :].reshape(1, 1, 1) for bb in range(B)],
            axis=0)
        mask = (obj3 >= theta3).astype(jnp.float32)
        cnt_f = jnp.sum(mask, axis=2, keepdims=True)

        def _moments(t):
            acts3 = acts_s[t]
            s1 = jnp.sum(acts3 * mask, axis=2, keepdims=True)
            mean3 = s1 / cnt_f
            return acts3, mean3

        @pl.when(it < ITERS - 1)
        def _refit_all():
            for t in range(T):
                acts3, mean3 = _moments(t)
                dev = acts3 - jnp.broadcast_to(mean3, (B, A, C))
                s2 = jnp.sum(dev * dev * mask, axis=2, keepdims=True)
                mean_s[:, t] = mean3
                std_s[:, t] = jnp.sqrt(s2 / cnt_f)

        @pl.when(it == ITERS - 1)
        def _emit():
            _, mean3 = _moments(0)
            out_ref[...] = mean3


def _run(beliefT, WbT, WaT, WsT, wrT, eps2):
    return pl.pallas_call(
        _cem_kernel,
        grid=(ITERS, G),
        in_specs=[
            pl.BlockSpec((B, H, 1), lambda it, ci: (0, 0, 0)),
            pl.BlockSpec((H, H), lambda it, ci: (0, 0)),
            pl.BlockSpec((H, A), lambda it, ci: (0, 0)),
            pl.BlockSpec((Z, H), lambda it, ci: (0, 0)),
            pl.BlockSpec((1, H + Z), lambda it, ci: (0, 0)),
            pl.BlockSpec((1, T, A, 1, 1, NC),
                         lambda it, ci: (it, 0, 0, ci, 0, 0)),
        ],
        out_specs=pl.BlockSpec((B, A, 1), lambda it, ci: (0, 0, 0)),
        out_shape=jax.ShapeDtypeStruct((B, A, 1), jnp.float32),
        scratch_shapes=[
            pltpu.VMEM((G, 1, NC), jnp.float32),
            pltpu.VMEM((T, B, A, C), jnp.float32),
            pltpu.VMEM((B, T, A, 1), jnp.float32),
            pltpu.VMEM((B, T, A, 1), jnp.float32),
        ],
    )(beliefT, WbT, WaT, WsT, wrT, eps2)


def kernel(belief, state, Wb, Wa, Ws, Wr):
    del state
    key = jax.random.key(42)
    eps = jnp.stack([
        jax.random.normal(jax.random.fold_in(key, i), (T, B, C, A),
                          dtype=jnp.float32)
        for i in range(ITERS)
    ])
    eps2 = jnp.transpose(eps, (0, 1, 4, 2, 3)).reshape(ITERS, T, A, G, 1, NC)
    out3 = _run(belief[:, :, None], Wb.T, Wa.T, Ws.T, Wr.T, eps2)
    return out3[:, :, 0]


# speedup vs baseline: 1.0718x; 1.0718x over previous
import jax
import jax.numpy as jnp
import numpy as np
from jax.experimental import pallas as pl
from jax.experimental.pallas import tpu as pltpu

B = 16
C = 1000
T = 12
A = 6
H = 128
Z = 32
N = B * C
TOPK = 100
ITERS = 2
MIN_A = -1.0
MAX_A = 1.0
G = 2
CB = B // G
NC = CB * C

_SIGN = np.array(1 << 31, dtype=np.uint32).view(np.int32).item()


def _fkey(x):
    return x ^ (jax.lax.shift_right_arithmetic(x, 31) & jnp.int32(0x7FFFFFFF))


def _cem_kernel(beliefT_ref, WbT_ref, WaT_ref, WsT_ref, wrT_ref,
                eps_ref, out_ref, objrow_s, acts_s, mean_s, std_s):
    it = pl.program_id(0)
    ci = pl.program_id(1)

    @pl.when(jnp.logical_and(it == 0, ci == 0))
    def _init():
        mean_s[...] = jnp.zeros_like(mean_s)
        std_s[...] = jnp.ones_like(std_s)

    WbT = WbT_ref[...]
    WaT = WaT_ref[...]
    WsT = WsT_ref[...]
    wrT = wrT_ref[...]

    b = jnp.concatenate(
        [jnp.broadcast_to(beliefT_ref[ci * CB + j], (H, C))
         for j in range(CB)], axis=1)
    objacc = jnp.zeros((1, NC), dtype=jnp.float32)
    for t in range(T):
        eps_t = eps_ref[0, t, :, 0, 0]
        pieces = [jnp.clip(
            jnp.broadcast_to(mean_s[ci * CB + j, t], (A, C))
            + jnp.broadcast_to(std_s[ci * CB + j, t], (A, C))
            * eps_t[:, j * C:(j + 1) * C],
            MIN_A, MAX_A) for j in range(CB)]
        for j in range(CB):
            acts_s[t, ci * CB + j] = pieces[j]
        act = jnp.concatenate(pieces, axis=1)
        pre = (jnp.dot(WbT, b, preferred_element_type=jnp.float32)
               + jnp.dot(WaT, act, preferred_element_type=jnp.float32))
        b = jnp.tanh(pre)
        s = jnp.tanh(jnp.dot(WsT, b, preferred_element_type=jnp.float32))
        feats = jnp.concatenate([b, s], axis=0)
        objacc = objacc + jnp.dot(wrT, feats,
                                  preferred_element_type=jnp.float32)
    objrow_s[ci] = objacc

    @pl.when(ci == G - 1)
    def _select_and_refit():
        obj2 = jnp.concatenate(
            [objrow_s[bb // CB, :, (bb % CB) * C:(bb % CB + 1) * C]
             for bb in range(B)], axis=0)
        obj3 = jnp.concatenate(
            [objrow_s[bb // CB, :, (bb % CB) * C:(bb % CB + 1) * C]
             .reshape(1, 1, C) for bb in range(B)], axis=0)

        key = _fkey(jax.lax.bitcast_convert_type(obj2, jnp.int32))
        theta_u = jnp.zeros((B, 1), dtype=jnp.int32)
        for bit in range(31, -1, -1):
            bit_val = np.array(1 << bit, dtype=np.uint32).view(np.int32).item()
            cand_u = theta_u | jnp.int32(bit_val)
            cand_s = cand_u ^ jnp.int32(_SIGN)
            cnt = jnp.sum((key >= cand_s).astype(jnp.int32), axis=1,
                          keepdims=True)
            theta_u = jnp.where(cnt >= TOPK, cand_u, theta_u)
        theta_f = jax.lax.bitcast_convert_type(
            _fkey(theta_u ^ jnp.int32(_SIGN)), jnp.float32)
        theta3 = jnp.concatenate(
            [theta_f[bb:bb + 1, :].reshape(1, 1, 1) for bb in range(B)],
            axis=0)
        mask = (obj3 >= theta3).astype(jnp.float32)
        cnt_f = jnp.sum(mask, axis=2, keepdims=True)

        def _moments(t):
            acts3 = acts_s[t]
            s1 = jnp.sum(acts3 * mask, axis=2, keepdims=True)
            mean3 = s1 / cnt_f
            return acts3, mean3

        @pl.when(it < ITERS - 1)
        def _refit_all():
            for t in range(T):
                acts3, mean3 = _moments(t)
                dev = acts3 - jnp.broadcast_to(mean3, (B, A, C))
                s2 = jnp.sum(dev * dev * mask, axis=2, keepdims=True)
                mean_s[:, t] = mean3
                std_s[:, t] = jnp.sqrt(s2 / cnt_f)

        @pl.when(it == ITERS - 1)
        def _emit():
            _, mean3 = _moments(0)
            out_ref[...] = mean3


def _run(beliefT, WbT, WaT, WsT, wrT, eps2):
    return pl.pallas_call(
        _cem_kernel,
        grid=(ITERS, G),
        in_specs=[
            pl.BlockSpec((B, H, 1), lambda it, ci: (0, 0, 0)),
            pl.BlockSpec((H, H), lambda it, ci: (0, 0)),
            pl.BlockSpec((H, A), lambda it, ci: (0, 0)),
            pl.BlockSpec((Z, H), lambda it, ci: (0, 0)),
            pl.BlockSpec((1, H + Z), lambda it, ci: (0, 0)),
            pl.BlockSpec((1, T, A, 1, 1, NC),
                         lambda it, ci: (it, 0, 0, ci, 0, 0)),
        ],
        out_specs=pl.BlockSpec((B, A, 1), lambda it, ci: (0, 0, 0)),
        out_shape=jax.ShapeDtypeStruct((B, A, 1), jnp.float32),
        scratch_shapes=[
            pltpu.VMEM((G, 1, NC), jnp.float32),
            pltpu.VMEM((T, B, A, C), jnp.float32),
            pltpu.VMEM((B, T, A, 1), jnp.float32),
            pltpu.VMEM((B, T, A, 1), jnp.float32),
        ],
    )(beliefT, WbT, WaT, WsT, wrT, eps2)


def kernel(belief, state, Wb, Wa, Ws, Wr):
    del state
    key = jax.random.key(42)
    eps = jnp.stack([
        jax.random.normal(jax.random.fold_in(key, i), (T, B, C, A),
                          dtype=jnp.float32)
        for i in range(ITERS)
    ])
    eps2 = jnp.transpose(eps, (0, 1, 4, 2, 3)).reshape(ITERS, T, A, G, 1, NC)
    out3 = _run(belief[:, :, None], Wb.T, Wa.T, Ws.T, Wr.T, eps2)
    return out3[:, :, 0]
